# baseline (speedup 1.0000x reference)
import jax
import jax.numpy as jnp
from jax import lax
from jax.experimental import pallas as pl
from jax.experimental.pallas import tpu as pltpu
from jax.experimental.pallas import tpu_sc as plsc

_INFO = plsc.get_sparse_core_info()
_NC = _INFO.num_cores
_NS = _INFO.num_subcores
_NW = _NC * _NS

_N = 1024
_S = 20
_B = _N * _S
_D = 2048
_V = 38
_BPW = _B // _NW
_K = 16
_TS = 3
_TD = _D // 128


def _body(table_hbm, idx_hbm, out_hbm, table_v, idx_v, *sems):
    wid = lax.axis_index("s") * _NC + lax.axis_index("c")
    base = wid * _BPW
    pltpu.sync_copy(idx_hbm.at[pl.ds(base, _BPW)], idx_v)
    pltpu.sync_copy(table_hbm, table_v)

    def row_start(q, ts, sr, v, b):
        pltpu.async_copy(table_v.at[v], out_hbm.at[q, ts, :, sr], sems[b])

    def row_wait(b):
        pltpu.make_async_copy(
            table_v.at[0], out_hbm.at[0, 0, :, 0], sems[b]
        ).wait()

    @pl.loop(0, _BPW, step=_K)
    def _(g):
        @pl.when(g > 0)
        def _():
            for b in range(_K):
                row_wait(b)

        vals = idx_v[pl.ds(g, _K)]
        f0 = base + g
        q = f0 // _S
        r = f0 - q * _S
        for b in range(_K):
            row_start(q, r // 8, r - (r // 8) * 8, vals[b], b)
            is_last = r == (_S - 1)
            q = q + jnp.where(is_last, 1, 0)
            r = jnp.where(is_last, 0, r + 1)

    for b in range(_K):
        row_wait(b)


def kernel(signal, table):
    idx = signal.reshape(-1).astype(jnp.int32)
    table3 = table.reshape(_V, _TD, 128)
    mesh = plsc.VectorSubcoreMesh(core_axis_name="c", subcore_axis_name="s")
    run = pl.kernel(
        _body,
        mesh=mesh,
        out_type=jax.ShapeDtypeStruct((_N, _TS, _TD, 8, 128), jnp.float32),
        scratch_types=[
            pltpu.VMEM((_V, _TD, 128), jnp.float32),
            pltpu.VMEM((_BPW,), jnp.int32),
        ]
        + [pltpu.SemaphoreType.DMA] * _K,
    )
    tiles = run(table3, idx)
    full = tiles.transpose(0, 1, 3, 2, 4).reshape(_N, _TS * 8, _TD * 128)
    return full[:, :_S, :]


# speedup vs baseline: 1.2012x; 1.2012x over previous
import functools

import jax
import jax.numpy as jnp
from jax import lax
from jax.experimental import pallas as pl
from jax.experimental.pallas import tpu as pltpu
from jax.experimental.pallas import tpu_sc as plsc

_INFO = plsc.get_sparse_core_info()
_NC = _INFO.num_cores
_NS = _INFO.num_subcores
_NW = _NC * _NS

_N = 1024
_S = 20
_B = _N * _S
_D = 2048
_V = 38
_BPW = _B // _NW
_K = 16
_TD = _D // 128

_QB = 8


def _sc_body(table_hbm, idx_hbm, out_hbm, table_v, idx_v, *sems):
    wid = lax.axis_index("s") * _NC + lax.axis_index("c")
    base = wid * _BPW
    pltpu.sync_copy(idx_hbm.at[pl.ds(base, _BPW)], idx_v)
    pltpu.sync_copy(table_hbm, table_v)

    def row_start(i, v, b):
        pltpu.async_copy(table_v.at[v], out_hbm.at[base + i], sems[b])

    def row_wait(b):
        pltpu.make_async_copy(table_v.at[0], out_hbm.at[0], sems[b]).wait()

    @pl.loop(0, _BPW, step=_K)
    def _(g):
        @pl.when(g > 0)
        def _():
            for b in range(_K):
                row_wait(b)

        vals = idx_v[pl.ds(g, _K)]
        for b in range(_K):
            row_start(g + b, vals[b], b)

    for b in range(_K):
        row_wait(b)


def _tc_body(rows_ref, out_ref):
    x = rows_ref[...]
    out_ref[...] = x.reshape(_QB, _S, _D)


def kernel(signal, table):
    idx = signal.reshape(-1).astype(jnp.int32)
    table3 = table.reshape(_V, _TD, 128)
    mesh = plsc.VectorSubcoreMesh(core_axis_name="c", subcore_axis_name="s")
    gather = pl.kernel(
        _sc_body,
        mesh=mesh,
        out_type=jax.ShapeDtypeStruct((_B, _TD, 128), jnp.float32),
        scratch_types=[
            pltpu.VMEM((_V, _TD, 128), jnp.float32),
            pltpu.VMEM((_BPW,), jnp.int32),
        ]
        + [pltpu.SemaphoreType.DMA] * _K,
    )
    rows = gather(table3, idx)

    relayout = pl.pallas_call(
        _tc_body,
        grid=(_N // _QB,),
        in_specs=[
            pl.BlockSpec((_QB * _S, _TD, 128), lambda q: (q, 0, 0)),
        ],
        out_specs=pl.BlockSpec((_QB, _S, _D), lambda q: (q, 0, 0)),
        out_shape=jax.ShapeDtypeStruct((_N, _S, _D), jnp.float32),
    )
    return relayout(rows)


# speedup vs baseline: 2.0412x; 1.6993x over previous
import jax
import jax.numpy as jnp
from jax import lax
from jax.experimental import pallas as pl
from jax.experimental.pallas import tpu as pltpu
from jax.experimental.pallas import tpu_sc as plsc

_INFO = plsc.get_sparse_core_info()
_NC = _INFO.num_cores
_NS = _INFO.num_subcores
_NW = _NC * _NS

_N = 1024
_S = 20
_B = _N * _S
_D = 2048
_V = 38
_BPW = _B // _NW
_K = 16


def _body(table_hbm, idx_hbm, out_hbm, table_v, idx_v, *sems):
    wid = lax.axis_index("s") * _NC + lax.axis_index("c")
    base = wid * _BPW
    pltpu.sync_copy(idx_hbm.at[pl.ds(base, _BPW)], idx_v)
    pltpu.sync_copy(table_hbm, table_v)

    def row_start(q, r, v, b):
        pltpu.async_copy(table_v.at[v], out_hbm.at[q, r], sems[b])

    def row_wait(b):
        pltpu.make_async_copy(table_v.at[0], out_hbm.at[0, 0], sems[b]).wait()

    @pl.loop(0, _BPW, step=_K)
    def _(g):
        @pl.when(g > 0)
        def _():
            for b in range(_K):
                row_wait(b)

        vals = idx_v[pl.ds(g, _K)]
        f0 = base + g
        q = f0 // _S
        r = f0 - q * _S
        for b in range(_K):
            row_start(q, r, vals[b], b)
            is_last = r == (_S - 1)
            q = q + jnp.where(is_last, 1, 0)
            r = jnp.where(is_last, 0, r + 1)

    for b in range(_K):
        row_wait(b)


def kernel(signal, table):
    idx = signal.reshape(-1).astype(jnp.int32)
    mesh = plsc.VectorSubcoreMesh(core_axis_name="c", subcore_axis_name="s")
    run = pl.kernel(
        _body,
        mesh=mesh,
        out_type=jax.ShapeDtypeStruct((_N, _S, _D), jnp.float32),
        scratch_types=[
            pltpu.VMEM((_V, _D), jnp.float32),
            pltpu.VMEM((_BPW,), jnp.int32),
        ]
        + [pltpu.SemaphoreType.DMA] * _K,
    )
    return run(table, idx)
